# baseline (speedup 1.0000x reference)
import jax
import jax.numpy as jnp
from jax import lax
from jax.experimental import pallas as pl
from jax.experimental.pallas import tpu as pltpu
from jax.experimental.pallas import tpu_sc as plsc

D = 64
B = 4096
L = 50
LAST_ROW = 1000000
HALF_ROWS = 500000
NC = 2
NS = 16
NW = NC * NS
BAGS_PER_W = B // NW
BAGS_PER_CHUNK = 2
IDX_PER_CHUNK = BAGS_PER_CHUNK * L
NCHUNK = BAGS_PER_W // BAGS_PER_CHUNK
LANES = 16
NDG = D // LANES
NBUF = 4


def _ebag_body(idx_hbm, w0_hbm, w1_hbm, t2, out_hbm,
               idx_v, w0_v, w1_v, rows_bufs, acc_v, sems):
    wid = lax.axis_index("s") * NC + lax.axis_index("c")
    crow = wid * NCHUNK
    pltpu.sync_copy(idx_hbm.at[pl.ds(crow, NCHUNK), :], idx_v)
    pltpu.sync_copy(w0_hbm.at[pl.ds(crow, NCHUNK), :], w0_v)
    pltpu.sync_copy(w1_hbm.at[pl.ds(crow, NCHUNK), :], w1_v)

    blocks = []
    off = 0
    while off + LANES <= L:
        blocks.append((off, off))
        off += LANES
    if off < L:
        blocks.append((L - LANES, off))

    def start_gather(c, buf):
        pltpu.async_copy(t2.at[idx_v.at[c]], rows_bufs[buf], sems[buf])

    def wait_gather(c, buf):
        pltpu.make_async_copy(
            t2.at[idx_v.at[c]], rows_bufs[buf], sems[buf]
        ).wait()

    def compute_chunk(c, buf):
        rows_v = rows_bufs[buf]
        for i in range(BAGS_PER_CHUNK):
            base = i * L
            accs = [jnp.zeros((LANES,), jnp.float32) for _ in range(NDG)]
            for load_off, first_l in blocks:
                wv0 = w0_v[c, pl.ds(base + load_off, LANES)]
                wv1 = w1_v[c, pl.ds(base + load_off, LANES)]
                for lane in range(first_l - load_off, LANES):
                    l = load_off + lane
                    w0 = wv0[lane]
                    w1 = wv1[lane]
                    for d in range(NDG):
                        accs[d] = (
                            accs[d]
                            + w0 * rows_v[base + l, pl.ds(d * LANES, LANES)]
                            + w1 * rows_v[base + l, pl.ds(D + d * LANES, LANES)]
                        )
            for d in range(NDG):
                acc_v[c * BAGS_PER_CHUNK + i, pl.ds(d * LANES, LANES)] = accs[d]

    for b in range(NBUF - 1):
        start_gather(b, b)

    def outer_body(g, carry):
        for b in range(NBUF):
            c = g * NBUF + b
            nxt = c + NBUF - 1
            nxt_buf = (b + NBUF - 1) % NBUF

            @pl.when(nxt < NCHUNK)
            def _():
                start_gather(nxt, nxt_buf)

            wait_gather(c, b)
            compute_chunk(c, b)
        return carry

    lax.fori_loop(0, NCHUNK // NBUF, outer_body, 0)
    pltpu.sync_copy(acc_v, out_hbm.at[pl.ds(wid * BAGS_PER_W, BAGS_PER_W), :])


RELAYOUT_BLK = 10000


def _relayout_body(i_ref, o_ref):
    j = pl.program_id(1)

    @pl.when(j == 0)
    def _():
        o_ref[:, 0:D] = i_ref[...]

    @pl.when(j == 1)
    def _():
        o_ref[:, D:2 * D] = i_ref[...]


def _pack_rows(table):
    nblk = HALF_ROWS // RELAYOUT_BLK
    return pl.pallas_call(
        _relayout_body,
        grid=(nblk, 2),
        in_specs=[
            pl.BlockSpec((RELAYOUT_BLK, D), lambda i, j, n=nblk: (i + j * n, 0)),
        ],
        out_specs=pl.BlockSpec((RELAYOUT_BLK, 2 * D), lambda i, j: (i, 0)),
        out_shape=jax.ShapeDtypeStruct((HALF_ROWS, 2 * D), jnp.float32),
    )(table)


def kernel(lookup_tensor, per_sample_weights, table):
    lk = lookup_tensor
    w = per_sample_weights * (1.0 / L)
    hi = (lk >= HALF_ROWS).astype(jnp.float32)
    safe = (lk < LAST_ROW).astype(jnp.float32)
    w0 = w * (1.0 - hi)
    w1 = w * hi * safe
    idxh = jnp.where(lk < LAST_ROW, lk % HALF_ROWS, 0).astype(jnp.int32)

    nrow = B // BAGS_PER_CHUNK
    idx2 = idxh.reshape(nrow, IDX_PER_CHUNK)
    w02 = w0.reshape(nrow, IDX_PER_CHUNK)
    w12 = w1.reshape(nrow, IDX_PER_CHUNK)

    mesh = plsc.VectorSubcoreMesh(
        core_axis_name="c", subcore_axis_name="s", num_cores=NC, num_subcores=NS
    )
    f = pl.kernel(
        _ebag_body,
        out_type=jax.ShapeDtypeStruct((B, D), jnp.float32),
        mesh=mesh,
        scratch_types=[
            pltpu.VMEM((NCHUNK, IDX_PER_CHUNK), jnp.int32),
            pltpu.VMEM((NCHUNK, IDX_PER_CHUNK), jnp.float32),
            pltpu.VMEM((NCHUNK, IDX_PER_CHUNK), jnp.float32),
            [pltpu.VMEM((IDX_PER_CHUNK, 2 * D), jnp.float32) for _ in range(NBUF)],
            pltpu.VMEM((BAGS_PER_W, D), jnp.float32),
            [pltpu.SemaphoreType.DMA for _ in range(NBUF)],
        ],
    )
    t2 = _pack_rows(table)
    out = f(idx2, w02, w12, t2)

    s = jnp.sum(jnp.where(lk == LAST_ROW, w, 0.0), axis=1)
    return out + s[:, None] * table[LAST_ROW]


# speedup vs baseline: 2.3938x; 2.3938x over previous
import jax
import jax.numpy as jnp
from jax import lax
from jax.experimental import pallas as pl
from jax.experimental.pallas import tpu as pltpu
from jax.experimental.pallas import tpu_sc as plsc

D = 64
B = 4096
L = 50
ROWS = 1000001
NC = 2
NS = 16
NW = NC * NS
BAGS_PER_W = B // NW
BAGS_PER_CHUNK = 2
IDX_PER_CHUNK = BAGS_PER_CHUNK * L
NCHUNK = BAGS_PER_W // BAGS_PER_CHUNK
LANES = 16
NDG = D // LANES
NBUF = 4


def _ebag_body(idx_hbm, w_hbm, t2, out_hbm, idx_v, w_v, rows_bufs, acc_v, sems):
    wid = lax.axis_index("s") * NC + lax.axis_index("c")
    crow = wid * NCHUNK
    pltpu.sync_copy(idx_hbm.at[pl.ds(crow, NCHUNK), :], idx_v)
    pltpu.sync_copy(w_hbm.at[pl.ds(crow, NCHUNK), :], w_v)

    blocks = []
    off = 0
    while off + LANES <= L:
        blocks.append((off, off))
        off += LANES
    if off < L:
        blocks.append((L - LANES, off))

    def start_gather(c, buf):
        pltpu.async_copy(t2.at[idx_v.at[c]], rows_bufs[buf], sems[buf])

    def wait_gather(c, buf):
        pltpu.make_async_copy(
            t2.at[idx_v.at[c]], rows_bufs[buf], sems[buf]
        ).wait()

    def compute_chunk(c, buf):
        rows_v = rows_bufs[buf]
        for i in range(BAGS_PER_CHUNK):
            base = i * L
            accs = [jnp.zeros((LANES,), jnp.float32) for _ in range(NDG)]
            for load_off, first_l in blocks:
                wv = w_v[c, pl.ds(base + load_off, LANES)]
                for lane in range(first_l - load_off, LANES):
                    l = load_off + lane
                    w = wv[lane]
                    for d in range(NDG):
                        accs[d] = accs[d] + w * rows_v[base + l, pl.ds(d * LANES, LANES)]
            for d in range(NDG):
                acc_v[c * BAGS_PER_CHUNK + i, pl.ds(d * LANES, LANES)] = accs[d]

    for b in range(NBUF - 1):
        start_gather(b, b)

    def outer_body(g, carry):
        for b in range(NBUF):
            c = g * NBUF + b
            nxt = c + NBUF - 1
            nxt_buf = (b + NBUF - 1) % NBUF

            @pl.when(nxt < NCHUNK)
            def _():
                start_gather(nxt, nxt_buf)

            wait_gather(c, b)
            compute_chunk(c, b)
        return carry

    lax.fori_loop(0, NCHUNK // NBUF, outer_body, 0)
    pltpu.sync_copy(acc_v, out_hbm.at[pl.ds(wid * BAGS_PER_W, BAGS_PER_W), :])


TR_BLK = 12800


def _transpose_body(i_ref, o_ref):
    o_ref[:, 0:D] = i_ref[...].T


def _rowmajor_table(table):
    nblk = (ROWS + TR_BLK - 1) // TR_BLK
    return pl.pallas_call(
        _transpose_body,
        grid=(nblk,),
        in_specs=[pl.BlockSpec((D, TR_BLK), lambda i: (0, i))],
        out_specs=pl.BlockSpec((TR_BLK, 2 * D), lambda i: (i, 0)),
        out_shape=jax.ShapeDtypeStruct((ROWS, 2 * D), jnp.float32),
    )(table.T)


def kernel(lookup_tensor, per_sample_weights, table):
    idx2 = lookup_tensor.reshape(B // BAGS_PER_CHUNK, IDX_PER_CHUNK)
    w2 = (per_sample_weights * (1.0 / L)).reshape(B // BAGS_PER_CHUNK, IDX_PER_CHUNK)

    mesh = plsc.VectorSubcoreMesh(
        core_axis_name="c", subcore_axis_name="s", num_cores=NC, num_subcores=NS
    )
    f = pl.kernel(
        _ebag_body,
        out_type=jax.ShapeDtypeStruct((B, D), jnp.float32),
        mesh=mesh,
        scratch_types=[
            pltpu.VMEM((NCHUNK, IDX_PER_CHUNK), jnp.int32),
            pltpu.VMEM((NCHUNK, IDX_PER_CHUNK), jnp.float32),
            [pltpu.VMEM((IDX_PER_CHUNK, 2 * D), jnp.float32) for _ in range(NBUF)],
            pltpu.VMEM((BAGS_PER_W, D), jnp.float32),
            [pltpu.SemaphoreType.DMA for _ in range(NBUF)],
        ],
    )
    t2 = _rowmajor_table(table)
    return f(idx2, w2, t2)


# speedup vs baseline: 2.4700x; 1.0318x over previous
import jax
import jax.numpy as jnp
from jax import lax
from jax.experimental import pallas as pl
from jax.experimental.pallas import tpu as pltpu
from jax.experimental.pallas import tpu_sc as plsc

D = 64
B = 4096
L = 50
ROWS = 1000001
NC = 2
NS = 16
NW = NC * NS
BAGS_PER_W = B // NW
BAGS_PER_CHUNK = 2
IDX_PER_CHUNK = BAGS_PER_CHUNK * L
NCHUNK = BAGS_PER_W // BAGS_PER_CHUNK
LANES = 16
NDG = D // LANES
NBUF = 4


def _ebag_body(idx_hbm, loc_hbm, w_hbm, t2, out_hbm,
               idx_v, loc_v, w_v, rows_bufs, acc_v, sems):
    wid = lax.axis_index("s") * NC + lax.axis_index("c")
    crow = wid * NCHUNK
    pltpu.sync_copy(idx_hbm.at[pl.ds(crow, NCHUNK), :], idx_v)
    pltpu.sync_copy(loc_hbm.at[pl.ds(crow, NCHUNK), :], loc_v)
    pltpu.sync_copy(w_hbm.at[pl.ds(crow, NCHUNK), :], w_v)

    blocks = []
    off = 0
    while off + LANES <= L:
        blocks.append((off, off))
        off += LANES
    if off < L:
        blocks.append((L - LANES, off))

    def start_gather(c, buf):
        pltpu.async_copy(t2.at[idx_v.at[c]], rows_bufs[buf], sems[buf])

    def wait_gather(c, buf):
        pltpu.make_async_copy(
            t2.at[idx_v.at[c]], rows_bufs[buf], sems[buf]
        ).wait()

    def compute_chunk(c, buf):
        rows_v = rows_bufs[buf]
        for i in range(BAGS_PER_CHUNK):
            base = i * L
            accs = [jnp.zeros((LANES,), jnp.float32) for _ in range(NDG)]
            for load_off, first_l in blocks:
                wv = w_v[c, pl.ds(base + load_off, LANES)]
                lv = loc_v[c, pl.ds(base + load_off, LANES)]
                for lane in range(first_l - load_off, LANES):
                    l = load_off + lane
                    w = wv[lane]
                    off = lv[lane]
                    for d in range(NDG):
                        accs[d] = accs[d] + w * rows_v[base + l, pl.ds(off + d * LANES, LANES)]
            for d in range(NDG):
                acc_v[c * BAGS_PER_CHUNK + i, pl.ds(d * LANES, LANES)] = accs[d]

    for b in range(NBUF - 1):
        start_gather(b, b)

    def outer_body(g, carry):
        for b in range(NBUF):
            c = g * NBUF + b
            nxt = c + NBUF - 1
            nxt_buf = (b + NBUF - 1) % NBUF

            @pl.when(nxt < NCHUNK)
            def _():
                start_gather(nxt, nxt_buf)

            wait_gather(c, b)
            compute_chunk(c, b)
        return carry

    lax.fori_loop(0, NCHUNK // NBUF, outer_body, 0)
    pltpu.sync_copy(acc_v, out_hbm.at[pl.ds(wid * BAGS_PER_W, BAGS_PER_W), :])


TR_BLK = 6400
TR_IN = 2 * TR_BLK
TR_NBLK = (ROWS + TR_IN - 1) // TR_IN
PACKED_ROWS = TR_NBLK * TR_BLK


def _transpose_body(i_ref, o_ref):
    y = i_ref[...].T
    o_ref[:, 0:D] = y[0:TR_BLK]
    o_ref[:, D:2 * D] = y[TR_BLK:TR_IN]


def _rowmajor_table(table):
    return pl.pallas_call(
        _transpose_body,
        grid=(TR_NBLK,),
        in_specs=[pl.BlockSpec((D, TR_IN), lambda i: (0, i))],
        out_specs=pl.BlockSpec((TR_BLK, 2 * D), lambda i: (i, 0)),
        out_shape=jax.ShapeDtypeStruct((PACKED_ROWS, 2 * D), jnp.float32),
    )(table.T)


def kernel(lookup_tensor, per_sample_weights, table):
    nrow = B // BAGS_PER_CHUNK
    g = lookup_tensor // TR_IN
    u = lookup_tensor % TR_IN
    idx2 = (g * TR_BLK + u % TR_BLK).astype(jnp.int32).reshape(nrow, IDX_PER_CHUNK)
    loc2 = ((u // TR_BLK) * D).astype(jnp.int32).reshape(nrow, IDX_PER_CHUNK)
    w2 = (per_sample_weights * (1.0 / L)).reshape(nrow, IDX_PER_CHUNK)

    mesh = plsc.VectorSubcoreMesh(
        core_axis_name="c", subcore_axis_name="s", num_cores=NC, num_subcores=NS
    )
    f = pl.kernel(
        _ebag_body,
        out_type=jax.ShapeDtypeStruct((B, D), jnp.float32),
        mesh=mesh,
        scratch_types=[
            pltpu.VMEM((NCHUNK, IDX_PER_CHUNK), jnp.int32),
            pltpu.VMEM((NCHUNK, IDX_PER_CHUNK), jnp.int32),
            pltpu.VMEM((NCHUNK, IDX_PER_CHUNK), jnp.float32),
            [pltpu.VMEM((IDX_PER_CHUNK, 2 * D), jnp.float32) for _ in range(NBUF)],
            pltpu.VMEM((BAGS_PER_W, D), jnp.float32),
            [pltpu.SemaphoreType.DMA for _ in range(NBUF)],
        ],
    )
    t2 = _rowmajor_table(table)
    return f(idx2, loc2, w2, t2)
